# baseline (speedup 1.0000x reference)
import functools

import jax
import jax.numpy as jnp
from jax import lax
from jax.experimental import pallas as pl
from jax.experimental.pallas import tpu as pltpu
from jax.experimental.pallas import tpu_sc as plsc

NC = 2
NS = 16
NW = NC * NS
LANES = 16

BAG = 50
CB = 2
CHUNK_TOK = CB * BAG
CHUNK_PAD = 112
NCLS = 4


def _tc_project(tabT, w4):
    V = tabT.shape[1]
    BL = 16384
    nblk = pl.cdiv(V, BL)
    VP = nblk * BL

    def body(t_ref, w_ref, *o_refs):
        p = jnp.dot(w_ref[...], t_ref[...], preferred_element_type=jnp.float32)
        for c, o_ref in enumerate(o_refs):
            o_ref[...] = p[c]

    return pl.pallas_call(
        body,
        grid=(nblk,),
        in_specs=[
            pl.BlockSpec((tabT.shape[0], BL), lambda i: (0, i)),
            pl.BlockSpec((NCLS, tabT.shape[0]), lambda i: (0, 0)),
        ],
        out_specs=[pl.BlockSpec((BL,), lambda i: (i,)) for _ in range(NCLS)],
        out_shape=[jax.ShapeDtypeStruct((VP,), jnp.float32)
                   for _ in range(NCLS)],
    )(tabT, w4)


def _allreduce16(x, lane):
    for sh in (1, 2, 4, 8):
        x = x + jnp.take(x, lane ^ sh)
    return x


def _sc_bag_sum(text_r, projs, bias16, *, batch):
    bags_per_w = batch // NW
    nchunks = bags_per_w // CB
    nk = CHUNK_PAD // LANES
    G = 4
    nsc = nchunks // G

    mesh = plsc.VectorSubcoreMesh(core_axis_name="c", subcore_axis_name="s")

    nbuf = 4
    assert nsc % nbuf == 0

    @functools.partial(
        pl.kernel,
        mesh=mesh,
        compiler_params=pltpu.CompilerParams(use_tc_tiling_on_sc=False),
        out_type=jax.ShapeDtypeStruct((batch, LANES), jnp.float32),
        scratch_types=[
            pltpu.VMEM((nsc, G * CHUNK_PAD), jnp.int32),
            pltpu.VMEM((nbuf, NCLS, G * CHUNK_PAD), jnp.float32),
            pltpu.VMEM((LANES,), jnp.float32),
            pltpu.VMEM((bags_per_w, LANES), jnp.float32),
        ] + [pltpu.SemaphoreType.DMA] * nbuf,
    )
    def kern(text_hbm, p0, p1, p2, p3, bias_hbm, out_hbm,
             idx_v, rows_v, bias_v, out_v, *sems):
        projs_hbm = (p0, p1, p2, p3)
        wid = lax.axis_index("s") * NC + lax.axis_index("c")
        pltpu.sync_copy(text_hbm.at[wid], idx_v)
        pltpu.sync_copy(bias_hbm, bias_v)
        bias = bias_v[...]
        lane = lax.iota(jnp.int32, LANES)
        inv = jnp.float32(1.0 / BAG)

        def start(ci, b):
            for c in range(NCLS):
                pltpu.async_copy(projs_hbm[c].at[idx_v.at[ci]],
                                 rows_v.at[b, c], sems[b])

        def wait(b):
            for c in range(NCLS):
                pltpu.make_async_copy(
                    projs_hbm[c].at[pl.ds(0, G * CHUNK_PAD)], rows_v.at[b, c],
                    sems[b]
                ).wait()

        for b in range(nbuf):
            start(b, b)

        def group_body(gi, carry):
            g0 = gi * nbuf
            for b in range(nbuf):
                si = g0 + b
                wait(b)
                for k2 in range(G):
                    base = k2 * CHUNK_PAD
                    redA = [None] * NCLS
                    redB = [None] * NCLS
                    for c in range(NCLS):
                        v = [rows_v[b, c, pl.ds(base + k * LANES, LANES)]
                             for k in range(nk)]
                        zero = jnp.zeros((LANES,), jnp.float32)
                        accA = v[0] + v[1] + v[2]
                        accA = accA + jnp.where(3 * LANES + lane < BAG,
                                                v[3], zero)
                        accB = jnp.where(3 * LANES + lane >= BAG, v[3], zero)
                        accB = accB + v[4] + v[5]
                        accB = accB + jnp.where(6 * LANES + lane < CHUNK_TOK,
                                                v[6], zero)
                        redA[c] = _allreduce16(accA, lane)
                        redB[c] = _allreduce16(accB, lane)
                    for bag, red in ((0, redA), (1, redB)):
                        r = jnp.zeros((LANES,), jnp.float32)
                        for c in range(NCLS):
                            r = jnp.where(lane == c, red[c], r)
                        out_v[(si * G + k2) * CB + bag] = r * inv + bias
                nsi = si + nbuf

                @pl.when(nsi < nsc)
                def _():
                    start(nsi, b)

            return carry

        lax.fori_loop(0, nsc // nbuf, group_body, 0)
        pltpu.sync_copy(out_v, out_hbm.at[pl.ds(wid * bags_per_w, bags_per_w)])

    return kern(text_r, *projs, bias16)


def kernel(text, offsets, table, fc_w, fc_b):
    batch = offsets.shape[0]
    bags_per_w = batch // NW
    nchunks = bags_per_w // CB
    text_r = text.astype(jnp.int32).reshape(NW, nchunks, CHUNK_TOK)
    text_r = jnp.pad(text_r, ((0, 0), (0, 0), (0, CHUNK_PAD - CHUNK_TOK)))
    text_r = text_r.reshape(NW, nchunks // 4, 4 * CHUNK_PAD)
    w4 = fc_w.astype(jnp.float32)
    bias16 = jnp.zeros((LANES,), jnp.float32).at[:NCLS].set(fc_b)
    projs = _tc_project(table.T, w4)
    out16 = _sc_bag_sum(text_r, projs, bias16, batch=batch)
    return out16[:, :4]


# speedup vs baseline: 1.1392x; 1.1392x over previous
import functools

import jax
import jax.numpy as jnp
from jax import lax
from jax.experimental import pallas as pl
from jax.experimental.pallas import tpu as pltpu
from jax.experimental.pallas import tpu_sc as plsc

NC = 2
NS = 16
NW = NC * NS
LANES = 16

BAG = 50
CB = 2
CHUNK_TOK = CB * BAG
CHUNK_PAD = 112
NCLS = 4


def _tc_project(tabT, w4):
    V = tabT.shape[1]
    BL = 16384
    nblk = pl.cdiv(V, BL)
    VP = nblk * BL

    def body(t_ref, w_ref, *o_refs):
        p = jnp.dot(w_ref[...], t_ref[...], preferred_element_type=jnp.float32)
        for c, o_ref in enumerate(o_refs):
            o_ref[...] = p[c]

    return pl.pallas_call(
        body,
        grid=(nblk,),
        in_specs=[
            pl.BlockSpec((tabT.shape[0], BL), lambda i: (0, i)),
            pl.BlockSpec((NCLS, tabT.shape[0]), lambda i: (0, 0)),
        ],
        out_specs=[pl.BlockSpec((BL,), lambda i: (i,)) for _ in range(NCLS)],
        out_shape=[jax.ShapeDtypeStruct((VP,), jnp.float32)
                   for _ in range(NCLS)],
    )(tabT, w4)


def _allreduce16(x, lane):
    for sh in (1, 2, 4, 8):
        x = x + jnp.take(x, lane ^ sh)
    return x


def _sc_bag_sum(text_r, projs, bias16, *, batch):
    bags_per_w = batch // NW
    nchunks = bags_per_w // CB
    nk = CHUNK_PAD // LANES

    mesh = plsc.VectorSubcoreMesh(core_axis_name="c", subcore_axis_name="s")

    nbuf = 16
    assert nchunks % nbuf == 0

    @functools.partial(
        pl.kernel,
        mesh=mesh,
        compiler_params=pltpu.CompilerParams(use_tc_tiling_on_sc=False),
        out_type=jax.ShapeDtypeStruct((batch, LANES), jnp.float32),
        scratch_types=[
            pltpu.VMEM((nchunks, CHUNK_PAD), jnp.int32),
            pltpu.VMEM((nbuf, NCLS, CHUNK_PAD), jnp.float32),
            pltpu.VMEM((LANES,), jnp.float32),
            pltpu.VMEM((bags_per_w, LANES), jnp.float32),
        ] + [pltpu.SemaphoreType.DMA] * nbuf,
    )
    def kern(text_hbm, p0, p1, p2, p3, bias_hbm, out_hbm,
             idx_v, rows_v, bias_v, out_v, *sems):
        projs_hbm = (p0, p1, p2, p3)
        wid = lax.axis_index("s") * NC + lax.axis_index("c")
        pltpu.sync_copy(text_hbm.at[wid], idx_v)
        pltpu.sync_copy(bias_hbm, bias_v)
        bias = bias_v[...]
        lane = lax.iota(jnp.int32, LANES)
        inv = jnp.float32(1.0 / BAG)

        def start(ci, b):
            for c in range(NCLS):
                pltpu.async_copy(projs_hbm[c].at[idx_v.at[ci]],
                                 rows_v.at[b, c], sems[b])

        def wait(b):
            for c in range(NCLS):
                pltpu.make_async_copy(
                    projs_hbm[c].at[pl.ds(0, CHUNK_PAD)], rows_v.at[b, c],
                    sems[b]
                ).wait()

        for b in range(nbuf):
            start(b, b)

        def group_body(gi, carry):
            g0 = gi * nbuf
            for b in range(nbuf):
                ci = g0 + b
                wait(b)
                redA = [None] * NCLS
                redB = [None] * NCLS
                for c in range(NCLS):
                    v = [rows_v[b, c, pl.ds(k * LANES, LANES)]
                         for k in range(nk)]
                    zero = jnp.zeros((LANES,), jnp.float32)
                    accA = v[0] + v[1] + v[2]
                    accA = accA + jnp.where(3 * LANES + lane < BAG, v[3], zero)
                    accB = jnp.where(3 * LANES + lane >= BAG, v[3], zero)
                    accB = accB + v[4] + v[5]
                    accB = accB + jnp.where(6 * LANES + lane < CHUNK_TOK,
                                            v[6], zero)
                    redA[c] = _allreduce16(accA, lane)
                    redB[c] = _allreduce16(accB, lane)
                for bag, red in ((0, redA), (1, redB)):
                    r = jnp.zeros((LANES,), jnp.float32)
                    for c in range(NCLS):
                        r = jnp.where(lane == c, red[c], r)
                    out_v[ci * CB + bag] = r * inv + bias
                nci = ci + nbuf

                @pl.when(nci < nchunks)
                def _():
                    start(nci, b)

            return carry

        lax.fori_loop(0, nchunks // nbuf, group_body, 0)
        pltpu.sync_copy(out_v, out_hbm.at[pl.ds(wid * bags_per_w, bags_per_w)])

    return kern(text_r, *projs, bias16)


def kernel(text, offsets, table, fc_w, fc_b):
    batch = offsets.shape[0]
    bags_per_w = batch // NW
    nchunks = bags_per_w // CB
    text_r = text.astype(jnp.int32).reshape(NW, nchunks, CHUNK_TOK)
    text_r = jnp.pad(text_r, ((0, 0), (0, 0), (0, CHUNK_PAD - CHUNK_TOK)))
    w4 = fc_w.astype(jnp.float32)
    bias16 = jnp.zeros((LANES,), jnp.float32).at[:NCLS].set(fc_b)
    projs = _tc_project(table.T, w4)
    out16 = _sc_bag_sum(text_r, projs, bias16, batch=batch)
    return out16[:, :4]
